# baseline (speedup 1.0000x reference)
import jax
import jax.numpy as jnp
from jax import lax
from jax.experimental import pallas as pl
from jax.experimental.pallas import tpu as pltpu
from jax.experimental.pallas import tpu_sc as plsc

_GAMMA = 12.0
_HID = 64
_TDIM = 32
_RDIM = 96
_B = 16384
_NW = 32
_NCHUNK = 4
_C = 128
_PER_W = _NCHUNK * _C

_S0 = 0.9999794
_S1 = -0.16662438
_S2 = 0.008308985
_S3 = -0.00019264995
_S4 = 2.1478727e-06
_INV2PI = 0.15915494309189535
_PI2_HI = 6.28125
_PI2_LO = 0.0019353071795864992
_RND = 12582912.0


def _sin(x):
    t = x * _INV2PI
    n = (t + _RND) - _RND
    y = (x - n * _PI2_HI) - n * _PI2_LO
    z = y * y
    p = _S4
    p = p * z + _S3
    p = p * z + _S2
    p = p * z + _S1
    p = p * z + _S0
    return p * y


def _make_kernel():
    mesh = plsc.VectorSubcoreMesh(core_axis_name="c", subcore_axis_name="s")

    def body(h_hbm, r_hbm, t_hbm, day_hbm, ent_hbm, rel_hbm, frq_hbm,
             phi_hbm, amp_hbm, out_hbm, hv, rv, tv, dayv, eh_v, et_v, rr_v,
             fh_v, ph_v, ah_v, ft_v, pt_v, at_v, scores_v, sem):
        wid = lax.axis_index("s") * 2 + lax.axis_index("c")
        pltpu.sync_copy(h_hbm.at[wid], hv)
        pltpu.sync_copy(r_hbm.at[wid], rv)
        pltpu.sync_copy(t_hbm.at[wid], tv)
        pltpu.sync_copy(day_hbm.at[wid], dayv)

        iota16 = lax.broadcasted_iota(jnp.int32, (16,), 0)

        for c in range(_NCHUNK):
            descs = [
                pltpu.async_copy(ent_hbm.at[hv.at[c]], eh_v, sem),
                pltpu.async_copy(ent_hbm.at[tv.at[c]], et_v, sem),
                pltpu.async_copy(rel_hbm.at[rv.at[c]], rr_v, sem),
                pltpu.async_copy(frq_hbm.at[hv.at[c]], fh_v, sem),
                pltpu.async_copy(phi_hbm.at[hv.at[c]], ph_v, sem),
                pltpu.async_copy(amp_hbm.at[hv.at[c]], ah_v, sem),
                pltpu.async_copy(frq_hbm.at[tv.at[c]], ft_v, sem),
                pltpu.async_copy(phi_hbm.at[tv.at[c]], pt_v, sem),
                pltpu.async_copy(amp_hbm.at[tv.at[c]], at_v, sem),
            ]
            for d in descs:
                d.wait()

            def group_body(g, _):
                row = g * 16 + iota16
                dv = dayv[c, pl.ds(g * 16, 16)]
                acc = jnp.zeros((16,), jnp.float32)

                def ent_f(fb, acc):
                    for u in range(4):
                        colf = jnp.full((16,), fb * 4 + u, jnp.int32)
                        eh = plsc.load_gather(eh_v, [row, colf])
                        et = plsc.load_gather(et_v, [row, colf])
                        rr = plsc.load_gather(rr_v, [row, colf])
                        acc = acc + jnp.abs(eh + rr - et)
                    return acc

                acc = lax.fori_loop(0, _HID // 4, ent_f, acc)

                def time_f(fb, acc):
                    for u in range(2):
                        colf = jnp.full((16,), fb * 2 + u, jnp.int32)
                        colr = colf + _HID
                        fh = plsc.load_gather(fh_v, [row, colf])
                        ph = plsc.load_gather(ph_v, [row, colf])
                        ah = plsc.load_gather(ah_v, [row, colf])
                        ft = plsc.load_gather(ft_v, [row, colf])
                        pt = plsc.load_gather(pt_v, [row, colf])
                        at = plsc.load_gather(at_v, [row, colf])
                        rr = plsc.load_gather(rr_v, [row, colr])
                        sh = _sin(dv * fh + ph)
                        st = _sin(dv * ft + pt)
                        acc = acc + jnp.abs(ah * sh + rr - at * st)
                    return acc

                acc = lax.fori_loop(0, _TDIM // 2, time_f, acc)
                scores_v[pl.ds(c * _C + g * 16, 16)] = _GAMMA - acc
                return 0

            lax.fori_loop(0, _C // 16, group_body, 0)

        pltpu.sync_copy(scores_v, out_hbm.at[wid])

    return pl.kernel(
        body,
        out_type=jax.ShapeDtypeStruct((_NW, _PER_W), jnp.float32),
        mesh=mesh,
        compiler_params=pltpu.CompilerParams(
            needs_layout_passes=False, use_tc_tiling_on_sc=False),
        scratch_types=[
            pltpu.VMEM((_NCHUNK, _C), jnp.int32),
            pltpu.VMEM((_NCHUNK, _C), jnp.int32),
            pltpu.VMEM((_NCHUNK, _C), jnp.int32),
            pltpu.VMEM((_NCHUNK, _C), jnp.float32),
            pltpu.VMEM((_C, _HID), jnp.float32),
            pltpu.VMEM((_C, _HID), jnp.float32),
            pltpu.VMEM((_C, _RDIM), jnp.float32),
            pltpu.VMEM((_C, _TDIM), jnp.float32),
            pltpu.VMEM((_C, _TDIM), jnp.float32),
            pltpu.VMEM((_C, _TDIM), jnp.float32),
            pltpu.VMEM((_C, _TDIM), jnp.float32),
            pltpu.VMEM((_C, _TDIM), jnp.float32),
            pltpu.VMEM((_C, _TDIM), jnp.float32),
            pltpu.VMEM((_PER_W,), jnp.float32),
            pltpu.SemaphoreType.DMA,
        ],
    )


_sc_kernel = _make_kernel()


def kernel(sample, entity_embedding, relation_embedding, d_frq_embedding,
           d_phi_embedding, d_amp_embedding):
    h = sample[:, 0].reshape(_NW, _NCHUNK, _C)
    r = sample[:, 1].reshape(_NW, _NCHUNK, _C)
    t = sample[:, 2].reshape(_NW, _NCHUNK, _C)
    day = sample[:, 3].astype(jnp.float32).reshape(_NW, _NCHUNK, _C)
    out = _sc_kernel(h, r, t, day, entity_embedding, relation_embedding,
                     d_frq_embedding, d_phi_embedding, d_amp_embedding)
    return out.reshape(_B, 1)


# speedup vs baseline: 1.0309x; 1.0309x over previous
import jax
import jax.numpy as jnp
from jax import lax
from jax.experimental import pallas as pl
from jax.experimental.pallas import tpu as pltpu
from jax.experimental.pallas import tpu_sc as plsc

_GAMMA = 12.0
_HID = 64
_TDIM = 32
_RDIM = 96
_B = 16384
_NW = 32
_NCHUNK = 4
_C = 128
_PER_W = _NCHUNK * _C

_S0 = 0.9999794
_S1 = -0.16662438
_S2 = 0.008308985
_S3 = -0.00019264995
_S4 = 2.1478727e-06
_INV2PI = 0.15915494309189535
_PI2_HI = 6.28125
_PI2_LO = 0.0019353071795864992
_RND = 12582912.0


def _sin(x):
    t = x * _INV2PI
    n = (t + _RND) - _RND
    y = (x - n * _PI2_HI) - n * _PI2_LO
    z = y * y
    p = _S4
    p = p * z + _S3
    p = p * z + _S2
    p = p * z + _S1
    p = p * z + _S0
    return p * y


def _make_kernel():
    mesh = plsc.VectorSubcoreMesh(core_axis_name="c", subcore_axis_name="s")

    def body(h_hbm, r_hbm, t_hbm, day_hbm, ent_hbm, rel_hbm, frq_hbm,
             phi_hbm, amp_hbm, out_hbm, hv, rv, tv, dayv, eh_v, et_v, rr_v,
             fh_v, ph_v, ah_v, ft_v, pt_v, at_v, scores_v, sem0, sem1):
        wid = lax.axis_index("s") * 2 + lax.axis_index("c")
        pltpu.sync_copy(h_hbm.at[wid], hv)
        pltpu.sync_copy(r_hbm.at[wid], rv)
        pltpu.sync_copy(t_hbm.at[wid], tv)
        pltpu.sync_copy(day_hbm.at[wid], dayv)

        iota16 = lax.broadcasted_iota(jnp.int32, (16,), 0)
        sems = [sem0, sem1]

        def issue(c):
            k = c % 2
            sem = sems[k]
            return [
                pltpu.async_copy(ent_hbm.at[hv.at[c]], eh_v.at[k], sem),
                pltpu.async_copy(ent_hbm.at[tv.at[c]], et_v.at[k], sem),
                pltpu.async_copy(rel_hbm.at[rv.at[c]], rr_v.at[k], sem),
                pltpu.async_copy(frq_hbm.at[hv.at[c]], fh_v.at[k], sem),
                pltpu.async_copy(phi_hbm.at[hv.at[c]], ph_v.at[k], sem),
                pltpu.async_copy(amp_hbm.at[hv.at[c]], ah_v.at[k], sem),
                pltpu.async_copy(frq_hbm.at[tv.at[c]], ft_v.at[k], sem),
                pltpu.async_copy(phi_hbm.at[tv.at[c]], pt_v.at[k], sem),
                pltpu.async_copy(amp_hbm.at[tv.at[c]], at_v.at[k], sem),
            ]

        pending = {0: issue(0)}
        for c in range(_NCHUNK):
            k = c % 2
            for d in pending.pop(c):
                d.wait()
            if c + 1 < _NCHUNK:
                pending[c + 1] = issue(c + 1)

            eh_c, et_c, rr_c = eh_v.at[k], et_v.at[k], rr_v.at[k]
            fh_c, ph_c, ah_c = fh_v.at[k], ph_v.at[k], ah_v.at[k]
            ft_c, pt_c, at_c = ft_v.at[k], pt_v.at[k], at_v.at[k]

            def group_body(g, _):
                row = g * 16 + iota16
                dv = dayv[c, pl.ds(g * 16, 16)]
                acc = jnp.zeros((16,), jnp.float32)

                def ent_f(fb, acc):
                    for u in range(4):
                        colf = jnp.full((16,), fb * 4 + u, jnp.int32)
                        eh = plsc.load_gather(eh_c, [row, colf])
                        et = plsc.load_gather(et_c, [row, colf])
                        rr = plsc.load_gather(rr_c, [row, colf])
                        acc = acc + jnp.abs(eh + rr - et)
                    return acc

                acc = lax.fori_loop(0, _HID // 4, ent_f, acc)

                def time_f(fb, acc):
                    for u in range(2):
                        colf = jnp.full((16,), fb * 2 + u, jnp.int32)
                        colr = colf + _HID
                        fh = plsc.load_gather(fh_c, [row, colf])
                        ph = plsc.load_gather(ph_c, [row, colf])
                        ah = plsc.load_gather(ah_c, [row, colf])
                        ft = plsc.load_gather(ft_c, [row, colf])
                        pt = plsc.load_gather(pt_c, [row, colf])
                        at = plsc.load_gather(at_c, [row, colf])
                        rr = plsc.load_gather(rr_c, [row, colr])
                        sh = _sin(dv * fh + ph)
                        st = _sin(dv * ft + pt)
                        acc = acc + jnp.abs(ah * sh + rr - at * st)
                    return acc

                acc = lax.fori_loop(0, _TDIM // 2, time_f, acc)
                scores_v[pl.ds(c * _C + g * 16, 16)] = _GAMMA - acc
                return 0

            lax.fori_loop(0, _C // 16, group_body, 0)

        pltpu.sync_copy(scores_v, out_hbm.at[wid])

    return pl.kernel(
        body,
        out_type=jax.ShapeDtypeStruct((_NW, _PER_W), jnp.float32),
        mesh=mesh,
        compiler_params=pltpu.CompilerParams(
            needs_layout_passes=False, use_tc_tiling_on_sc=False,
            disable_bounds_checks=True),
        scratch_types=[
            pltpu.VMEM((_NCHUNK, _C), jnp.int32),
            pltpu.VMEM((_NCHUNK, _C), jnp.int32),
            pltpu.VMEM((_NCHUNK, _C), jnp.int32),
            pltpu.VMEM((_NCHUNK, _C), jnp.float32),
            pltpu.VMEM((2, _C, _HID), jnp.float32),
            pltpu.VMEM((2, _C, _HID), jnp.float32),
            pltpu.VMEM((2, _C, _RDIM), jnp.float32),
            pltpu.VMEM((2, _C, _TDIM), jnp.float32),
            pltpu.VMEM((2, _C, _TDIM), jnp.float32),
            pltpu.VMEM((2, _C, _TDIM), jnp.float32),
            pltpu.VMEM((2, _C, _TDIM), jnp.float32),
            pltpu.VMEM((2, _C, _TDIM), jnp.float32),
            pltpu.VMEM((2, _C, _TDIM), jnp.float32),
            pltpu.VMEM((_PER_W,), jnp.float32),
            pltpu.SemaphoreType.DMA,
            pltpu.SemaphoreType.DMA,
        ],
    )


_sc_kernel = _make_kernel()


def kernel(sample, entity_embedding, relation_embedding, d_frq_embedding,
           d_phi_embedding, d_amp_embedding):
    h = sample[:, 0].reshape(_NW, _NCHUNK, _C)
    r = sample[:, 1].reshape(_NW, _NCHUNK, _C)
    t = sample[:, 2].reshape(_NW, _NCHUNK, _C)
    day = sample[:, 3].astype(jnp.float32).reshape(_NW, _NCHUNK, _C)
    out = _sc_kernel(h, r, t, day, entity_embedding, relation_embedding,
                     d_frq_embedding, d_phi_embedding, d_amp_embedding)
    return out.reshape(_B, 1)
